# baseline (speedup 1.0000x reference)
import functools

import jax
import jax.numpy as jnp
from jax import lax
from jax.experimental import pallas as pl
from jax.experimental.pallas import tpu as pltpu
from jax.experimental.pallas import tpu_sc as plsc

_EPS = 1e-5
_NC = 2
_NS = 16
_K = 128
_CA = 56
_CB = 48


def _sc_gather_scatter_add(x, src2d, dst2d, zeros, n_pad):
    d = x.shape[1]
    rows_per_tile = n_pad // _NS

    @functools.partial(
        pl.kernel,
        out_type=jax.ShapeDtypeStruct((_NC, n_pad, d), jnp.float32),
        mesh=plsc.VectorSubcoreMesh(core_axis_name="c", subcore_axis_name="s"),
        scratch_types=[
            pltpu.VMEM_SHARED((n_pad, d), jnp.float32),
            pltpu.VMEM((_CA, _K), jnp.int32),
            pltpu.VMEM((_CA, _K), jnp.int32),
            pltpu.VMEM((_CB, _K), jnp.int32),
            pltpu.VMEM((_CB, _K), jnp.int32),
            pltpu.VMEM((_K, d), jnp.float32),
            pltpu.SemaphoreType.DMA,
        ],
    )
    def kern(x_hbm, src_hbm, dst_hbm, zeros_hbm, out_hbm, acc, srca_v, dsta_v,
             srcb_v, dstb_v, rows_v, sem):
        c = lax.axis_index("c")
        s = lax.axis_index("s")
        w = c * _NS + s
        pltpu.sync_copy(
            zeros_hbm.at[pl.ds(s * rows_per_tile, rows_per_tile)],
            acc.at[pl.ds(s * rows_per_tile, rows_per_tile)],
        )
        pltpu.sync_copy(src_hbm.at[pl.ds(w * _CA, _CA)], srca_v)
        pltpu.sync_copy(dst_hbm.at[pl.ds(w * _CA, _CA)], dsta_v)

        @pl.when(c == 0)
        def _():
            base_b = _NC * _NS * _CA + s * _CB
            pltpu.sync_copy(src_hbm.at[pl.ds(base_b, _CB)], srcb_v)
            pltpu.sync_copy(dst_hbm.at[pl.ds(base_b, _CB)], dstb_v)

        plsc.subcore_barrier()

        def body_a(j, carry):
            pltpu.async_copy(x_hbm.at[srca_v.at[j]], rows_v, sem).wait()
            pltpu.sync_copy(rows_v, acc.at[dsta_v.at[j]], add=True)
            return carry

        lax.fori_loop(0, _CA, body_a, 0)

        @pl.when(c == 0)
        def _():
            def body_b(j, carry):
                pltpu.async_copy(x_hbm.at[srcb_v.at[j]], rows_v, sem).wait()
                pltpu.sync_copy(rows_v, acc.at[dstb_v.at[j]], add=True)
                return carry

            lax.fori_loop(0, _CB, body_b, 0)

        plsc.subcore_barrier()
        pltpu.sync_copy(
            acc.at[pl.ds(s * rows_per_tile, rows_per_tile)],
            out_hbm.at[c, pl.ds(s * rows_per_tile, rows_per_tile)],
        )

    return kern(x, src2d, dst2d, zeros)


def _tc_dense(agg, x, w_l, b_l, wg0, wg1, bg, asp, gamma, beta, blk):
    n_pad, d = x.shape

    def body(agg_ref, x_ref, w_ref, b_ref, wg0_ref, wg1_ref, bg_ref, asp_ref,
             g_ref, be_ref, o_ref):
        a = agg_ref[0] + agg_ref[1]
        h = jnp.dot(a, w_ref[...], preferred_element_type=jnp.float32) + b_ref[...]
        h = jnp.maximum(h, 0.0)
        gc = jnp.dot(asp_ref[...], wg1_ref[...], preferred_element_type=jnp.float32) + bg_ref[...]
        gate = jax.nn.sigmoid(
            jnp.dot(h, wg0_ref[...], preferred_element_type=jnp.float32) + gc)
        xn = gate * h + (1.0 - gate) * x_ref[...]
        mu = jnp.mean(xn, axis=-1, keepdims=True)
        var = jnp.mean((xn - mu) * (xn - mu), axis=-1, keepdims=True)
        o_ref[...] = (xn - mu) * lax.rsqrt(var + _EPS) * g_ref[...] + be_ref[...]

    full = lambda i: (0, 0)
    return pl.pallas_call(
        body,
        grid=(n_pad // blk,),
        in_specs=[
            pl.BlockSpec((_NC, blk, d), lambda i: (0, i, 0)),
            pl.BlockSpec((blk, d), lambda i: (i, 0)),
            pl.BlockSpec((d, d), full),
            pl.BlockSpec((1, d), full),
            pl.BlockSpec((d, d), full),
            pl.BlockSpec((d, d), full),
            pl.BlockSpec((1, d), full),
            pl.BlockSpec((1, d), full),
            pl.BlockSpec((1, d), full),
            pl.BlockSpec((1, d), full),
        ],
        out_specs=pl.BlockSpec((blk, d), lambda i: (i, 0)),
        out_shape=jax.ShapeDtypeStruct((n_pad, d), jnp.float32),
    )(agg, x, w_l, b_l, wg0, wg1, bg, asp, gamma, beta)


def kernel(token_embeddings, edge_index, aspect_embedding, W0, b0, W1, b1,
           Wg, bg, ln_g0, ln_b0, ln_g1, ln_b1):
    n, d = token_embeddings.shape
    e = edge_index.shape[1]
    nw = _NC * _NS

    blk = 1024
    n_pad = ((n + 1 + blk - 1) // blk) * blk
    assert (n_pad // _NS) % 8 == 0

    n_rows = _NC * _NS * _CA + _NS * _CB
    assert n_rows * _K >= e
    e_pad = n_rows * _K
    src_p = jnp.zeros((e_pad,), jnp.int32).at[:e].set(edge_index[0])
    dst_p = jnp.full((e_pad,), n, jnp.int32).at[:e].set(edge_index[1])
    src2d = src_p.reshape(n_rows, _K)
    dst2d = dst_p.reshape(n_rows, _K)

    zeros = jnp.zeros((n_pad, d), jnp.float32)
    x = jnp.zeros((n_pad, d), jnp.float32).at[:n].set(token_embeddings)
    wg0 = Wg[:d]
    wg1 = Wg[d:]
    asp = aspect_embedding.reshape(1, d)
    bg2 = bg.reshape(1, d)

    for (w_l, b_l, g_l, be_l) in ((W0, b0, ln_g0, ln_b0), (W1, b1, ln_g1, ln_b1)):
        agg = _sc_gather_scatter_add(x, src2d, dst2d, zeros, n_pad)
        x = _tc_dense(agg, x, w_l, b_l.reshape(1, d), wg0, wg1, bg2, asp,
                      g_l.reshape(1, d), be_l.reshape(1, d), blk)
    return x[:n]


# speedup vs baseline: 1.8709x; 1.8709x over previous
import functools

import jax
import jax.numpy as jnp
from jax import lax
from jax.experimental import pallas as pl
from jax.experimental.pallas import tpu as pltpu
from jax.experimental.pallas import tpu_sc as plsc

_EPS = 1e-5
_NC = 2
_NS = 16
_K = 128


def _sc_gather_scatter_add(x, src3d, dst3d, zeros, n_pad):
    d = x.shape[1]
    n_chunks = src3d.shape[1]
    rows_per_tile = n_pad // _NS

    @functools.partial(
        pl.kernel,
        out_type=jax.ShapeDtypeStruct((_NC, n_pad, d), jnp.float32),
        mesh=plsc.VectorSubcoreMesh(core_axis_name="c", subcore_axis_name="s"),
        scratch_types=[
            pltpu.VMEM_SHARED((n_pad, d), jnp.float32),
            pltpu.VMEM((n_chunks, _K), jnp.int32),
            pltpu.VMEM((n_chunks, _K), jnp.int32),
            pltpu.VMEM((_K, d), jnp.float32),
            pltpu.SemaphoreType.DMA,
        ],
    )
    def kern(x_hbm, src_hbm, dst_hbm, zeros_hbm, out_hbm, acc, src_v, dst_v, rows_v, sem):
        c = lax.axis_index("c")
        s = lax.axis_index("s")
        w = c * _NS + s
        pltpu.sync_copy(
            zeros_hbm.at[pl.ds(s * rows_per_tile, rows_per_tile)],
            acc.at[pl.ds(s * rows_per_tile, rows_per_tile)],
        )
        pltpu.sync_copy(src_hbm.at[w], src_v)
        pltpu.sync_copy(dst_hbm.at[w], dst_v)
        plsc.subcore_barrier()

        def body(j, carry):
            pltpu.async_copy(x_hbm.at[src_v.at[j]], rows_v, sem).wait()
            pltpu.sync_copy(rows_v, acc.at[dst_v.at[j]], add=True)
            return carry

        lax.fori_loop(0, n_chunks, body, 0)
        plsc.subcore_barrier()
        pltpu.sync_copy(
            acc.at[pl.ds(s * rows_per_tile, rows_per_tile)],
            out_hbm.at[c, pl.ds(s * rows_per_tile, rows_per_tile)],
        )

    return kern(x, src3d, dst3d, zeros)


def _tc_dense(agg, x, w_l, b_l, wg0, wg1, bg, asp, gamma, beta, blk):
    n_pad, d = x.shape

    def body(agg_ref, x_ref, w_ref, b_ref, wg0_ref, wg1_ref, bg_ref, asp_ref,
             g_ref, be_ref, o_ref):
        a = agg_ref[0] + agg_ref[1]
        h = jnp.dot(a, w_ref[...], preferred_element_type=jnp.float32) + b_ref[...]
        h = jnp.maximum(h, 0.0)
        gc = jnp.dot(asp_ref[...], wg1_ref[...], preferred_element_type=jnp.float32) + bg_ref[...]
        gate = jax.nn.sigmoid(
            jnp.dot(h, wg0_ref[...], preferred_element_type=jnp.float32) + gc)
        xn = gate * h + (1.0 - gate) * x_ref[...]
        mu = jnp.mean(xn, axis=-1, keepdims=True)
        var = jnp.mean((xn - mu) * (xn - mu), axis=-1, keepdims=True)
        o_ref[...] = (xn - mu) * lax.rsqrt(var + _EPS) * g_ref[...] + be_ref[...]

    full = lambda i: (0, 0)
    return pl.pallas_call(
        body,
        grid=(n_pad // blk,),
        in_specs=[
            pl.BlockSpec((_NC, blk, d), lambda i: (0, i, 0)),
            pl.BlockSpec((blk, d), lambda i: (i, 0)),
            pl.BlockSpec((d, d), full),
            pl.BlockSpec((1, d), full),
            pl.BlockSpec((d, d), full),
            pl.BlockSpec((d, d), full),
            pl.BlockSpec((1, d), full),
            pl.BlockSpec((1, d), full),
            pl.BlockSpec((1, d), full),
            pl.BlockSpec((1, d), full),
        ],
        out_specs=pl.BlockSpec((blk, d), lambda i: (i, 0)),
        out_shape=jax.ShapeDtypeStruct((n_pad, d), jnp.float32),
    )(agg, x, w_l, b_l, wg0, wg1, bg, asp, gamma, beta)


def kernel(token_embeddings, edge_index, aspect_embedding, W0, b0, W1, b1,
           Wg, bg, ln_g0, ln_b0, ln_g1, ln_b1):
    n, d = token_embeddings.shape
    e = edge_index.shape[1]
    nw = _NC * _NS

    blk = 1024
    n_pad = ((n + 1 + blk - 1) // blk) * blk
    assert (n_pad // _NS) % 8 == 0

    e_per_tile = -(-e // nw)
    n_chunks = -(-e_per_tile // _K)
    e_pad = nw * n_chunks * _K
    src_p = jnp.zeros((e_pad,), jnp.int32).at[:e].set(edge_index[0])
    dst_p = jnp.full((e_pad,), n, jnp.int32).at[:e].set(edge_index[1])
    src3d = src_p.reshape(nw, n_chunks, _K)
    dst3d = dst_p.reshape(nw, n_chunks, _K)

    zeros = jnp.zeros((n_pad, d), jnp.float32)
    x = jnp.zeros((n_pad, d), jnp.float32).at[:n].set(token_embeddings)
    wg0 = Wg[:d]
    wg1 = Wg[d:]
    asp = aspect_embedding.reshape(1, d)
    bg2 = bg.reshape(1, d)

    for (w_l, b_l, g_l, be_l) in ((W0, b0, ln_g0, ln_b0), (W1, b1, ln_g1, ln_b1)):
        agg = _sc_gather_scatter_add(x, src3d, dst3d, zeros, n_pad)
        x = _tc_dense(agg, x, w_l, b_l.reshape(1, d), wg0, wg1, bg2, asp,
                      g_l.reshape(1, d), be_l.reshape(1, d), blk)
    return x[:n]


# speedup vs baseline: 1.8867x; 1.0084x over previous
import functools

import jax
import jax.numpy as jnp
from jax import lax
from jax.experimental import pallas as pl
from jax.experimental.pallas import tpu as pltpu
from jax.experimental.pallas import tpu_sc as plsc

_EPS = 1e-5
_NC = 2
_NS = 16
_K = 128


def _sc_gather_scatter_add(x, src3d, dst3d, zeros, n_pad):
    d = x.shape[1]
    n_chunks = src3d.shape[1]
    rows_per_tile = n_pad // _NS

    @functools.partial(
        pl.kernel,
        out_type=jax.ShapeDtypeStruct((_NC, n_pad, d), jnp.float32),
        mesh=plsc.VectorSubcoreMesh(core_axis_name="c", subcore_axis_name="s"),
        scratch_types=[
            pltpu.VMEM_SHARED((n_pad, d), jnp.float32),
            pltpu.VMEM((n_chunks, _K), jnp.int32),
            pltpu.VMEM((n_chunks, _K), jnp.int32),
            pltpu.VMEM((_K, d), jnp.float32),
            pltpu.SemaphoreType.DMA,
        ],
    )
    def kern(x_hbm, src_hbm, dst_hbm, zeros_hbm, out_hbm, acc, src_v, dst_v, rows_v, sem):
        c = lax.axis_index("c")
        s = lax.axis_index("s")
        w = c * _NS + s
        pltpu.sync_copy(
            zeros_hbm.at[pl.ds(s * rows_per_tile, rows_per_tile)],
            acc.at[pl.ds(s * rows_per_tile, rows_per_tile)],
        )
        pltpu.sync_copy(src_hbm.at[w], src_v)
        pltpu.sync_copy(dst_hbm.at[w], dst_v)
        plsc.subcore_barrier()

        def body(j, carry):
            pltpu.async_copy(x_hbm.at[src_v.at[j]], rows_v, sem).wait()
            pltpu.sync_copy(rows_v, acc.at[dst_v.at[j]], add=True)
            return carry

        lax.fori_loop(0, n_chunks, body, 0)
        plsc.subcore_barrier()
        pltpu.sync_copy(
            acc.at[pl.ds(s * rows_per_tile, rows_per_tile)],
            out_hbm.at[c, pl.ds(s * rows_per_tile, rows_per_tile)],
        )

    return kern(x, src3d, dst3d, zeros)


def _tc_dense(agg, x, w_l, b_l, wg0, wg1, bg, asp, gamma, beta, blk):
    n_pad, d = x.shape

    def body(agg_ref, x_ref, w_ref, b_ref, wg0_ref, wg1_ref, bg_ref, asp_ref,
             g_ref, be_ref, o_ref):
        a = agg_ref[0] + agg_ref[1]
        h = jnp.dot(a, w_ref[...], preferred_element_type=jnp.float32) + b_ref[...]
        h = jnp.maximum(h, 0.0)
        gc = jnp.dot(asp_ref[...], wg1_ref[...], preferred_element_type=jnp.float32) + bg_ref[...]
        gate = jax.nn.sigmoid(
            jnp.dot(h, wg0_ref[...], preferred_element_type=jnp.float32) + gc)
        xn = gate * h + (1.0 - gate) * x_ref[...]
        mu = jnp.mean(xn, axis=-1, keepdims=True)
        var = jnp.mean((xn - mu) * (xn - mu), axis=-1, keepdims=True)
        o_ref[...] = (xn - mu) * lax.rsqrt(var + _EPS) * g_ref[...] + be_ref[...]

    full = lambda i: (0, 0)
    return pl.pallas_call(
        body,
        grid=(n_pad // blk,),
        in_specs=[
            pl.BlockSpec((_NC, blk, d), lambda i: (0, i, 0)),
            pl.BlockSpec((blk, d), lambda i: (i, 0)),
            pl.BlockSpec((d, d), full),
            pl.BlockSpec((1, d), full),
            pl.BlockSpec((d, d), full),
            pl.BlockSpec((d, d), full),
            pl.BlockSpec((1, d), full),
            pl.BlockSpec((1, d), full),
            pl.BlockSpec((1, d), full),
            pl.BlockSpec((1, d), full),
        ],
        out_specs=pl.BlockSpec((blk, d), lambda i: (i, 0)),
        out_shape=jax.ShapeDtypeStruct((n_pad, d), jnp.float32),
    )(agg, x, w_l, b_l, wg0, wg1, bg, asp, gamma, beta)


def kernel(token_embeddings, edge_index, aspect_embedding, W0, b0, W1, b1,
           Wg, bg, ln_g0, ln_b0, ln_g1, ln_b1):
    n, d = token_embeddings.shape
    e = edge_index.shape[1]
    nw = _NC * _NS

    blk = 2048
    n_pad = ((n + 1 + blk - 1) // blk) * blk
    assert (n_pad // _NS) % 8 == 0

    e_per_tile = -(-e // nw)
    n_chunks = -(-e_per_tile // _K)
    e_pad = nw * n_chunks * _K
    src_p = jnp.zeros((e_pad,), jnp.int32).at[:e].set(edge_index[0])
    dst_p = jnp.full((e_pad,), n, jnp.int32).at[:e].set(edge_index[1])
    src3d = src_p.reshape(nw, n_chunks, _K)
    dst3d = dst_p.reshape(nw, n_chunks, _K)

    zeros = jnp.zeros((n_pad, d), jnp.float32)
    x = jnp.zeros((n_pad, d), jnp.float32).at[:n].set(token_embeddings)
    wg0 = Wg[:d]
    wg1 = Wg[d:]
    asp = aspect_embedding.reshape(1, d)
    bg2 = bg.reshape(1, d)

    for (w_l, b_l, g_l, be_l) in ((W0, b0, ln_g0, ln_b0), (W1, b1, ln_g1, ln_b1)):
        agg = _sc_gather_scatter_add(x, src3d, dst3d, zeros, n_pad)
        x = _tc_dense(agg, x, w_l, b_l.reshape(1, d), wg0, wg1, bg2, asp,
                      g_l.reshape(1, d), be_l.reshape(1, d), blk)
    return x[:n]
